# speedup vs baseline: 1.2772x; 1.2772x over previous
import jax
import jax.numpy as jnp
from jax.experimental import pallas as pl

_ROWS = 2048


def _fused_kernel(x_ref, t_ref, temp_ref,
                  te_w1_ref, te_b1_ref, te_w2_ref, te_b2_ref,
                  alpha_w_ref, alpha_b_ref, beta_w_ref, beta_b_ref,
                  gw1_t_ref, gw1_e_ref, gate_b1_ref, gate_w2_ref, gate_b2_ref,
                  k_vec_ref, es_w_ref, es_b_ref, ee_w_ref, ee_b_ref,
                  bl_w_ref, bl_b_ref,
                  out_ref, base_ref, tempc_ref, event_ref, seas_ref, gate_ref):
    xb = x_ref[...]
    t_norm = t_ref[...] * (1.0 / 168.0)
    tempb = temp_ref[...]

    h = jnp.maximum(tempb * te_w1_ref[...] + te_b1_ref[...], 0.0)
    temp_embed = jnp.dot(h, te_w2_ref[...],
                         preferred_element_type=jnp.float32) + te_b2_ref[...]

    alpha_k = jnp.dot(temp_embed, alpha_w_ref[...],
                      preferred_element_type=jnp.float32) + alpha_b_ref[...]
    beta_k = jnp.dot(temp_embed, beta_w_ref[...],
                     preferred_element_type=jnp.float32) + beta_b_ref[...]
    harmonics = (2.0 * jnp.pi) * t_norm * k_vec_ref[...]
    seasonal = jnp.sum(alpha_k * jnp.sin(harmonics)
                       + beta_k * jnp.cos(harmonics),
                       axis=1, keepdims=True)

    gh = jnp.maximum(t_norm * gw1_t_ref[...]
                     + jnp.dot(temp_embed, gw1_e_ref[...],
                               preferred_element_type=jnp.float32)
                     + gate_b1_ref[...], 0.0)
    gate = jax.nn.sigmoid(jnp.dot(gh, gate_w2_ref[...],
                                  preferred_element_type=jnp.float32)
                          + gate_b2_ref[...])
    temp_component = gate * seasonal

    baseline = jnp.dot(xb, bl_w_ref[...],
                       preferred_element_type=jnp.float32) + bl_b_ref[...]

    scores = jnp.dot(xb, es_w_ref[...],
                     preferred_element_type=jnp.float32) + es_b_ref[...]
    lanes = jax.lax.broadcasted_iota(jnp.int32, scores.shape, 1)
    m1 = jnp.max(scores, axis=1, keepdims=True)
    i1 = jnp.min(jnp.where(scores == m1, lanes, scores.shape[1]),
                 axis=1, keepdims=True)
    mask1 = lanes == i1
    rest = jnp.where(mask1, -jnp.inf, scores)
    m2 = jnp.max(rest, axis=1, keepdims=True)
    i2 = jnp.min(jnp.where(rest == m2, lanes, scores.shape[1]),
                 axis=1, keepdims=True)
    selected = jnp.where(mask1 | (lanes == i2), scores, 0.0)
    event = (jnp.sum(selected * ee_w_ref[...], axis=1, keepdims=True)
             + ee_b_ref[...])

    out_ref[...] = baseline + temp_component + event
    base_ref[...] = baseline
    tempc_ref[...] = temp_component
    event_ref[...] = event
    seas_ref[...] = seasonal
    gate_ref[...] = gate


@jax.jit
def kernel(x, t, temp, te_w1, te_b1, te_w2, te_b2, alpha_w, alpha_b,
           beta_w, beta_b, gate_w1, gate_b1, gate_w2, gate_b2, k_vector,
           es_w, es_b, ee_w, ee_b, bl_w, bl_b):
    B = x.shape[0]
    R = _ROWS
    grid = (B // R,)

    te_b1_2 = te_b1.reshape(1, -1)
    te_b2_2 = te_b2.reshape(1, -1)
    alpha_b_2 = alpha_b.reshape(1, -1)
    beta_b_2 = beta_b.reshape(1, -1)
    gw1_t = gate_w1[0:1, :]
    gw1_e = gate_w1[1:, :]
    gate_b1_2 = gate_b1.reshape(1, -1)
    gate_b2_2 = gate_b2.reshape(1, -1)
    es_b_2 = es_b.reshape(1, -1)
    ee_w_2 = ee_w.reshape(1, -1)
    ee_b_2 = ee_b.reshape(1, -1)
    bl_b_2 = bl_b.reshape(1, -1)

    def rows(shape):
        return pl.BlockSpec((R, shape[1]), lambda i: (i, 0))

    def whole(a):
        return pl.BlockSpec(a.shape, lambda i: (0, 0))

    small = [te_w1, te_b1_2, te_w2, te_b2_2, alpha_w, alpha_b_2, beta_w,
             beta_b_2, gw1_t, gw1_e, gate_b1_2, gate_w2, gate_b2_2,
             k_vector, es_w, es_b_2, ee_w_2, ee_b_2, bl_w, bl_b_2]

    out_shape = tuple(jax.ShapeDtypeStruct((B, 1), jnp.float32)
                      for _ in range(6))
    out_specs = tuple(pl.BlockSpec((R, 1), lambda i: (i, 0))
                      for _ in range(6))

    return pl.pallas_call(
        _fused_kernel,
        grid=grid,
        in_specs=[rows(x.shape), rows(t.shape), rows(temp.shape)]
                 + [whole(a) for a in small],
        out_specs=out_specs,
        out_shape=out_shape,
    )(x, t, temp, *small)


# speedup vs baseline: 5.0429x; 3.9482x over previous
import jax
import jax.numpy as jnp
from jax.experimental import pallas as pl

_ROWS = 2048
_LANES = 128


def _fused_kernel(x_ref, t_ref, temp_ref,
                  te_w1_ref, te_b1_ref, te_w2_ref, te_b2_ref,
                  alpha_w_ref, alpha_b_ref, beta_w_ref, beta_b_ref,
                  gw1_t_ref, gw1_e_ref, gate_b1_ref, gate_w2_ref, gate_b2_ref,
                  k_vec_ref, es_w_ref, es_b_ref, ee_w_ref, ee_b_ref,
                  bl_w_ref, bl_b_ref,
                  out_ref, base_ref, tempc_ref, event_ref, seas_ref, gate_ref):
    sub = _ROWS // _LANES
    tn = t_ref[...] * (1.0 / 168.0)
    tp = temp_ref[...]

    h = [jnp.maximum(tp * te_w1_ref[0, j] + te_b1_ref[0, j], 0.0)
         for j in range(16)]
    te = [te_b2_ref[0, k] + sum(h[j] * te_w2_ref[j, k] for j in range(16))
          for k in range(10)]

    seasonal = jnp.zeros_like(tn)
    for c in range(4):
        alpha_c = alpha_b_ref[0, c] + sum(te[k] * alpha_w_ref[k, c]
                                          for k in range(10))
        beta_c = beta_b_ref[0, c] + sum(te[k] * beta_w_ref[k, c]
                                        for k in range(10))
        harm_c = (2.0 * jnp.pi) * k_vec_ref[0, c] * tn
        seasonal = seasonal + alpha_c * jnp.sin(harm_c) + beta_c * jnp.cos(harm_c)

    gacc = gate_b2_ref[0, 0]
    gate = jnp.zeros_like(tn)
    for j in range(16):
        gh_j = jnp.maximum(tn * gw1_t_ref[0, j]
                           + sum(te[k] * gw1_e_ref[k, j] for k in range(10))
                           + gate_b1_ref[0, j], 0.0)
        gate = gate + gh_j * gate_w2_ref[j, 0]
    gate = jax.nn.sigmoid(gate + gacc)
    temp_component = gate * seasonal

    xb = x_ref[...]
    dn_t = (((0,), (1,)), ((), ()))
    scores = jax.lax.dot_general(es_w_ref[...], xb, dn_t,
                                 preferred_element_type=jnp.float32)
    scores = scores + es_b_ref[...]
    baseline = jax.lax.dot_general(bl_w_ref[...], xb, dn_t,
                                   preferred_element_type=jnp.float32)
    baseline = baseline + bl_b_ref[0, 0]

    rows = jax.lax.broadcasted_iota(jnp.int32, scores.shape, 0)
    m1 = jnp.max(scores, axis=0, keepdims=True)
    i1 = jnp.min(jnp.where(scores == m1, rows, scores.shape[0]),
                 axis=0, keepdims=True)
    mask1 = rows == i1
    rest = jnp.where(mask1, -jnp.inf, scores)
    m2 = jnp.max(rest, axis=0, keepdims=True)
    i2 = jnp.min(jnp.where(rest == m2, rows, scores.shape[0]),
                 axis=0, keepdims=True)
    selected = jnp.where(mask1 | (rows == i2), scores, 0.0)
    event = (jnp.sum(selected * ee_w_ref[...], axis=0, keepdims=True)
             + ee_b_ref[0, 0])

    base_d = baseline.reshape(sub, _LANES)
    event_d = event.reshape(sub, _LANES)

    out_ref[...] = base_d + temp_component + event_d
    base_ref[...] = base_d
    tempc_ref[...] = temp_component
    event_ref[...] = event_d
    seas_ref[...] = seasonal
    gate_ref[...] = gate


@jax.jit
def kernel(x, t, temp, te_w1, te_b1, te_w2, te_b2, alpha_w, alpha_b,
           beta_w, beta_b, gate_w1, gate_b1, gate_w2, gate_b2, k_vector,
           es_w, es_b, ee_w, ee_b, bl_w, bl_b):
    B = x.shape[0]
    R = _ROWS
    grid = (B // R,)
    sub = R // _LANES
    BD = B // _LANES

    t2 = t.reshape(BD, _LANES)
    temp2 = temp.reshape(BD, _LANES)

    te_b1_2 = te_b1.reshape(1, -1)
    te_b2_2 = te_b2.reshape(1, -1)
    alpha_b_2 = alpha_b.reshape(1, -1)
    beta_b_2 = beta_b.reshape(1, -1)
    gw1_t = gate_w1[0:1, :]
    gw1_e = gate_w1[1:, :]
    gate_b1_2 = gate_b1.reshape(1, -1)
    gate_b2_2 = gate_b2.reshape(1, -1)
    es_b_2 = es_b.reshape(-1, 1)
    ee_w_2 = ee_w.reshape(-1, 1)
    ee_b_2 = ee_b.reshape(1, -1)
    bl_b_2 = bl_b.reshape(1, -1)

    def whole(a):
        return pl.BlockSpec(a.shape, lambda i: (0, 0))

    small = [te_w1, te_b1_2, te_w2, te_b2_2, alpha_w, alpha_b_2, beta_w,
             beta_b_2, gw1_t, gw1_e, gate_b1_2, gate_w2, gate_b2_2,
             k_vector, es_w, es_b_2, ee_w_2, ee_b_2, bl_w, bl_b_2]

    dense_spec = pl.BlockSpec((sub, _LANES), lambda i: (i, 0))
    out_shape = tuple(jax.ShapeDtypeStruct((BD, _LANES), jnp.float32)
                      for _ in range(6))
    out_specs = tuple(dense_spec for _ in range(6))

    outs = pl.pallas_call(
        _fused_kernel,
        grid=grid,
        in_specs=[pl.BlockSpec((R, x.shape[1]), lambda i: (i, 0)),
                  dense_spec, dense_spec] + [whole(a) for a in small],
        out_specs=out_specs,
        out_shape=out_shape,
    )(x, t2, temp2, *small)

    return tuple(o.reshape(B, 1) for o in outs)


# speedup vs baseline: 5.5362x; 1.0978x over previous
import jax
import jax.numpy as jnp
from jax.experimental import pallas as pl

_ROWS = 4096
_LANES = 128


def _fused_kernel(x_ref, t_ref, temp_ref,
                  te_w1_ref, te_b1_ref, te_w2_ref, te_b2_ref,
                  alpha_w_ref, alpha_b_ref, beta_w_ref, beta_b_ref,
                  gw1_t_ref, gw1_e_ref, gate_b1_ref, gate_w2_ref, gate_b2_ref,
                  k_vec_ref, es_w_ref, es_b_ref, ee_w_ref, ee_b_ref,
                  bl_w_ref, bl_b_ref,
                  out_ref, base_ref, tempc_ref, event_ref, seas_ref, gate_ref):
    sub = _ROWS // _LANES
    tn = t_ref[...] * (1.0 / 168.0)
    tp = temp_ref[...]

    h = [jnp.maximum(tp * te_w1_ref[0, j] + te_b1_ref[0, j], 0.0)
         for j in range(16)]
    te = [te_b2_ref[0, k] + sum(h[j] * te_w2_ref[j, k] for j in range(16))
          for k in range(10)]

    seasonal = jnp.zeros_like(tn)
    for c in range(4):
        alpha_c = alpha_b_ref[0, c] + sum(te[k] * alpha_w_ref[k, c]
                                          for k in range(10))
        beta_c = beta_b_ref[0, c] + sum(te[k] * beta_w_ref[k, c]
                                        for k in range(10))
        harm_c = (2.0 * jnp.pi) * k_vec_ref[0, c] * tn
        seasonal = seasonal + alpha_c * jnp.sin(harm_c) + beta_c * jnp.cos(harm_c)

    gacc = gate_b2_ref[0, 0]
    gate = jnp.zeros_like(tn)
    for j in range(16):
        gh_j = jnp.maximum(tn * gw1_t_ref[0, j]
                           + sum(te[k] * gw1_e_ref[k, j] for k in range(10))
                           + gate_b1_ref[0, j], 0.0)
        gate = gate + gh_j * gate_w2_ref[j, 0]
    gate = jax.nn.sigmoid(gate + gacc)
    temp_component = gate * seasonal

    xb = x_ref[...]
    dn_t = (((0,), (1,)), ((), ()))
    scores = jax.lax.dot_general(es_w_ref[...], xb, dn_t,
                                 preferred_element_type=jnp.float32)
    scores = scores + es_b_ref[...]
    baseline = jax.lax.dot_general(bl_w_ref[...], xb, dn_t,
                                   preferred_element_type=jnp.float32)
    baseline = baseline + bl_b_ref[0, 0]

    rows = jax.lax.broadcasted_iota(jnp.int32, scores.shape, 0)
    m1 = jnp.max(scores, axis=0, keepdims=True)
    i1 = jnp.min(jnp.where(scores == m1, rows, scores.shape[0]),
                 axis=0, keepdims=True)
    mask1 = rows == i1
    rest = jnp.where(mask1, -jnp.inf, scores)
    m2 = jnp.max(rest, axis=0, keepdims=True)
    i2 = jnp.min(jnp.where(rest == m2, rows, scores.shape[0]),
                 axis=0, keepdims=True)
    selected = jnp.where(mask1 | (rows == i2), scores, 0.0)
    event = (jnp.sum(selected * ee_w_ref[...], axis=0, keepdims=True)
             + ee_b_ref[0, 0])

    base_d = baseline.reshape(sub, _LANES)
    event_d = event.reshape(sub, _LANES)

    out_ref[...] = base_d + temp_component + event_d
    base_ref[...] = base_d
    tempc_ref[...] = temp_component
    event_ref[...] = event_d
    seas_ref[...] = seasonal
    gate_ref[...] = gate


@jax.jit
def kernel(x, t, temp, te_w1, te_b1, te_w2, te_b2, alpha_w, alpha_b,
           beta_w, beta_b, gate_w1, gate_b1, gate_w2, gate_b2, k_vector,
           es_w, es_b, ee_w, ee_b, bl_w, bl_b):
    B = x.shape[0]
    R = _ROWS
    grid = (B // R,)
    sub = R // _LANES
    BD = B // _LANES

    t2 = t.reshape(BD, _LANES)
    temp2 = temp.reshape(BD, _LANES)

    te_b1_2 = te_b1.reshape(1, -1)
    te_b2_2 = te_b2.reshape(1, -1)
    alpha_b_2 = alpha_b.reshape(1, -1)
    beta_b_2 = beta_b.reshape(1, -1)
    gw1_t = gate_w1[0:1, :]
    gw1_e = gate_w1[1:, :]
    gate_b1_2 = gate_b1.reshape(1, -1)
    gate_b2_2 = gate_b2.reshape(1, -1)
    es_b_2 = es_b.reshape(-1, 1)
    ee_w_2 = ee_w.reshape(-1, 1)
    ee_b_2 = ee_b.reshape(1, -1)
    bl_b_2 = bl_b.reshape(1, -1)

    def whole(a):
        return pl.BlockSpec(a.shape, lambda i: (0, 0))

    small = [te_w1, te_b1_2, te_w2, te_b2_2, alpha_w, alpha_b_2, beta_w,
             beta_b_2, gw1_t, gw1_e, gate_b1_2, gate_w2, gate_b2_2,
             k_vector, es_w, es_b_2, ee_w_2, ee_b_2, bl_w, bl_b_2]

    dense_spec = pl.BlockSpec((sub, _LANES), lambda i: (i, 0))
    out_shape = tuple(jax.ShapeDtypeStruct((BD, _LANES), jnp.float32)
                      for _ in range(6))
    out_specs = tuple(dense_spec for _ in range(6))

    outs = pl.pallas_call(
        _fused_kernel,
        grid=grid,
        in_specs=[pl.BlockSpec((R, x.shape[1]), lambda i: (i, 0)),
                  dense_spec, dense_spec] + [whole(a) for a in small],
        out_specs=out_specs,
        out_shape=out_shape,
    )(x, t2, temp2, *small)

    return tuple(o.reshape(B, 1) for o in outs)
